# baseline (speedup 1.0000x reference)
import jax
import jax.numpy as jnp
from jax.experimental import pallas as pl
from jax.experimental.pallas import tpu as pltpu


def _upconv_kernel(x_ref, w0_ref, w1_ref, g_ref, b_ref, o_ref):
    h_in = x_ref.shape[1] - 2
    cout = o_ref.shape[0]

    w0 = w0_ref[...]
    w1 = w1_ref[...]
    g0 = g_ref[0]
    g1 = g_ref[1]
    g2 = g_ref[2]
    bias = b_ref[...]

    def row_pair(a, carry):
        xa = x_ref[:, a, :].astype(jnp.bfloat16)
        xb = x_ref[:, a + 1, :].astype(jnp.bfloat16)
        xc = x_ref[:, a + 2, :].astype(jnp.bfloat16)

        x20 = jnp.concatenate([xa, xb], axis=0)
        x21 = jnp.concatenate([xb, xc], axis=0)

        for py, w, x2 in ((0, w0, x20), (1, w1, x21)):
            m = jnp.dot(w, x2, preferred_element_type=jnp.float32)
            mb = m.astype(jnp.bfloat16)
            y = jnp.dot(mb[0:cout], g0, preferred_element_type=jnp.float32)
            y = y + jnp.dot(mb[cout:2 * cout], g1,
                            preferred_element_type=jnp.float32)
            y = y + jnp.dot(mb[2 * cout:3 * cout], g2,
                            preferred_element_type=jnp.float32)
            y = y + bias
            o_ref[:, 2 * a + py, :] = y.astype(o_ref.dtype)
        return carry

    jax.lax.fori_loop(0, h_in, row_pair, None)


def kernel(x_nchw, weight, bias):
    n, cin, h, w = x_nchw.shape
    cout = weight.shape[0]
    ho, wo = 2 * h, 2 * w
    wp2 = w + 2

    x_pad = jnp.pad(x_nchw, ((0, 0), (0, 0), (1, 1), (1, 1)))

    def pack(wa, wb):
        wc = jnp.stack([wa, wb], axis=2)
        return jnp.transpose(wc, (3, 0, 2, 1)).reshape(3 * cout, 2 * cin)

    w0_mat = pack(weight[:, :, 0, :], weight[:, :, 1, :] + weight[:, :, 2, :])
    w1_mat = pack(weight[:, :, 0, :] + weight[:, :, 1, :], weight[:, :, 2, :])
    w0_mat = w0_mat.astype(jnp.bfloat16)
    w1_mat = w1_mat.astype(jnp.bfloat16)

    ow_idx = jnp.arange(wo)
    g_list = []
    for kx in range(3):
        j = ow_idx + kx
        src = jnp.where(j == 0, 0,
                        jnp.where(j == wo + 1, w + 1, (j - 1) // 2 + 1))
        g = (jnp.arange(wp2)[:, None] == src[None, :])
        g_list.append(g)
    g_all = jnp.stack(g_list, axis=0).astype(jnp.bfloat16)

    bias2d = bias.reshape(cout, 1)

    return pl.pallas_call(
        _upconv_kernel,
        out_shape=jax.ShapeDtypeStruct((n, cout, ho, wo), x_nchw.dtype),
        grid_spec=pltpu.PrefetchScalarGridSpec(
            num_scalar_prefetch=0,
            grid=(n,),
            in_specs=[
                pl.BlockSpec((None, cin, h + 2, wp2), lambda b: (b, 0, 0, 0)),
                pl.BlockSpec((3 * cout, 2 * cin), lambda b: (0, 0)),
                pl.BlockSpec((3 * cout, 2 * cin), lambda b: (0, 0)),
                pl.BlockSpec((3, wp2, wo), lambda b: (0, 0, 0)),
                pl.BlockSpec((cout, 1), lambda b: (0, 0)),
            ],
            out_specs=pl.BlockSpec((None, cout, ho, wo), lambda b: (b, 0, 0, 0)),
        ),
        compiler_params=pltpu.CompilerParams(
            dimension_semantics=("parallel",),
            vmem_limit_bytes=64 * 1024 * 1024,
        ),
    )(x_pad, w0_mat, w1_mat, g_all, bias2d)


# speedup vs baseline: 2.8849x; 2.8849x over previous
import jax
import jax.numpy as jnp
from jax.experimental import pallas as pl
from jax.experimental.pallas import tpu as pltpu

_RB = 8


def _upconv_kernel(xa_ref, xb_ref, w0_ref, w1_ref, g_ref, b_ref, o_ref):
    cin = xa_ref.shape[0]
    cout = o_ref.shape[0]

    rows = [xa_ref[:, r, :] for r in range(_RB)] + [xb_ref[:, 0, :],
                                                    xb_ref[:, 1, :]]
    x10 = jnp.concatenate(rows, axis=0)

    a_kx = [
        jnp.dot(x10, g_ref[kx],
                preferred_element_type=jnp.float32).astype(jnp.bfloat16)
        for kx in range(3)
    ]

    w0 = w0_ref[...]
    w1 = w1_ref[...]
    bias = b_ref[...]

    def hcat(r0, r1):
        return jnp.concatenate(
            [a_kx[0][r0 * cin:(r0 + 1) * cin],
             a_kx[1][r0 * cin:(r0 + 1) * cin],
             a_kx[2][r0 * cin:(r0 + 1) * cin],
             a_kx[0][r1 * cin:(r1 + 1) * cin],
             a_kx[1][r1 * cin:(r1 + 1) * cin],
             a_kx[2][r1 * cin:(r1 + 1) * cin]], axis=0)

    for k in range(_RB):
        for py, w in ((0, w0), (1, w1)):
            h = hcat(k + py, k + 1 + py)
            y = jnp.dot(w, h, preferred_element_type=jnp.float32) + bias
            o_ref[:, 2 * k + py, :] = y.astype(o_ref.dtype)


def kernel(x_nchw, weight, bias):
    n, cin, h, w = x_nchw.shape
    cout = weight.shape[0]
    ho, wo = 2 * h, 2 * w
    wp2 = w + 2
    nblk = h // _RB
    hp = (nblk + 1) * _RB

    x_pad = jnp.pad(x_nchw,
                    ((0, 0), (0, 0), (1, hp - h - 1), (1, 1))
                    ).astype(jnp.bfloat16)

    def pack(wa, wb):
        t = jnp.stack([jnp.transpose(wa, (0, 2, 1)),
                       jnp.transpose(wb, (0, 2, 1))], axis=1)
        return t.reshape(cout, 6 * cin).astype(jnp.bfloat16)

    wk = weight
    w0_mat = pack(wk[:, :, 0, :], wk[:, :, 1, :] + wk[:, :, 2, :])
    w1_mat = pack(wk[:, :, 0, :] + wk[:, :, 1, :], wk[:, :, 2, :])

    ow_idx = jnp.arange(wo)
    g_list = []
    for kx in range(3):
        j = ow_idx + kx
        src = jnp.where(j == 0, 0,
                        jnp.where(j == wo + 1, w + 1, (j - 1) // 2 + 1))
        g_list.append(jnp.arange(wp2)[:, None] == src[None, :])
    g_all = jnp.stack(g_list, axis=0).astype(jnp.bfloat16)

    bias2d = bias.reshape(cout, 1)

    return pl.pallas_call(
        _upconv_kernel,
        out_shape=jax.ShapeDtypeStruct((n, cout, ho, wo), x_nchw.dtype),
        grid_spec=pltpu.PrefetchScalarGridSpec(
            num_scalar_prefetch=0,
            grid=(n, nblk),
            in_specs=[
                pl.BlockSpec((None, cin, _RB, wp2), lambda b, r: (b, 0, r, 0)),
                pl.BlockSpec((None, cin, _RB, wp2),
                             lambda b, r: (b, 0, r + 1, 0)),
                pl.BlockSpec((cout, 6 * cin), lambda b, r: (0, 0)),
                pl.BlockSpec((cout, 6 * cin), lambda b, r: (0, 0)),
                pl.BlockSpec((3, wp2, wo), lambda b, r: (0, 0, 0)),
                pl.BlockSpec((cout, 1), lambda b, r: (0, 0)),
            ],
            out_specs=pl.BlockSpec((None, cout, 2 * _RB, wo),
                                   lambda b, r: (b, 0, r, 0)),
        ),
        compiler_params=pltpu.CompilerParams(
            dimension_semantics=("parallel", "arbitrary"),
            vmem_limit_bytes=100 * 1024 * 1024,
        ),
    )(x_pad, x_pad, w0_mat, w1_mat, g_all, bias2d)


# speedup vs baseline: 3.4794x; 1.2061x over previous
import jax
import jax.numpy as jnp
from jax.experimental import pallas as pl
from jax.experimental.pallas import tpu as pltpu

_RB = 8


def _upconv_kernel(xa_ref, xb_ref, w0_ref, w1_ref, g_ref, b_ref, o_ref):
    cin = xa_ref.shape[1]
    wp2 = xa_ref.shape[2]
    cout = o_ref.shape[0]

    x10 = jnp.concatenate(
        [xa_ref[...].reshape(_RB * cin, wp2),
         xb_ref[0:2].reshape(2 * cin, wp2)], axis=0)

    a_kx = [
        jnp.dot(x10, g_ref[kx],
                preferred_element_type=jnp.float32).astype(jnp.bfloat16)
        for kx in range(3)
    ]

    w0 = w0_ref[...]
    w1 = w1_ref[...]
    bias = b_ref[...]

    for k in range(_RB):
        for py, w in ((0, w0), (1, w1)):
            base = (k + py) * cin
            y = bias
            for kx in range(3):
                y = y + jnp.dot(w[:, kx * 2 * cin:(kx + 1) * 2 * cin],
                                a_kx[kx][base:base + 2 * cin],
                                preferred_element_type=jnp.float32)
            o_ref[:, 2 * k + py, :] = y.astype(o_ref.dtype)


def kernel(x_nchw, weight, bias):
    n, cin, h, w = x_nchw.shape
    cout = weight.shape[0]
    ho, wo = 2 * h, 2 * w
    wp2 = w + 2
    nblk = h // _RB
    hp = (nblk + 1) * _RB

    x_pad = jnp.pad(x_nchw,
                    ((0, 0), (0, 0), (1, hp - h - 1), (1, 1))
                    ).astype(jnp.bfloat16)
    x_t = jnp.transpose(x_pad, (0, 2, 1, 3))

    def pack(wa, wb):
        t = jnp.stack([jnp.transpose(wa, (0, 2, 1)),
                       jnp.transpose(wb, (0, 2, 1))], axis=2)
        return t.reshape(cout, 6 * cin).astype(jnp.bfloat16)

    wk = weight
    w0_mat = pack(wk[:, :, 0, :], wk[:, :, 1, :] + wk[:, :, 2, :])
    w1_mat = pack(wk[:, :, 0, :] + wk[:, :, 1, :], wk[:, :, 2, :])

    ow_idx = jnp.arange(wo)
    g_list = []
    for kx in range(3):
        j = ow_idx + kx
        src = jnp.where(j == 0, 0,
                        jnp.where(j == wo + 1, w + 1, (j - 1) // 2 + 1))
        g_list.append(jnp.arange(wp2)[:, None] == src[None, :])
    g_all = jnp.stack(g_list, axis=0).astype(jnp.bfloat16)

    bias2d = bias.reshape(cout, 1)

    return pl.pallas_call(
        _upconv_kernel,
        out_shape=jax.ShapeDtypeStruct((n, cout, ho, wo), x_nchw.dtype),
        grid_spec=pltpu.PrefetchScalarGridSpec(
            num_scalar_prefetch=0,
            grid=(n, nblk),
            in_specs=[
                pl.BlockSpec((None, _RB, cin, wp2), lambda b, r: (b, r, 0, 0)),
                pl.BlockSpec((None, _RB, cin, wp2),
                             lambda b, r: (b, r + 1, 0, 0)),
                pl.BlockSpec((cout, 6 * cin), lambda b, r: (0, 0)),
                pl.BlockSpec((cout, 6 * cin), lambda b, r: (0, 0)),
                pl.BlockSpec((3, wp2, wo), lambda b, r: (0, 0, 0)),
                pl.BlockSpec((cout, 1), lambda b, r: (0, 0)),
            ],
            out_specs=pl.BlockSpec((None, cout, 2 * _RB, wo),
                                   lambda b, r: (b, 0, r, 0)),
        ),
        compiler_params=pltpu.CompilerParams(
            dimension_semantics=("parallel", "parallel"),
            vmem_limit_bytes=100 * 1024 * 1024,
        ),
    )(x_t, x_t, w0_mat, w1_mat, g_all, bias2d)
